# baseline (speedup 1.0000x reference)
import jax
import jax.numpy as jnp
from jax import lax
from jax.experimental import pallas as pl
from jax.experimental.pallas import tpu as pltpu
from jax.experimental.pallas import tpu_sc as plsc

N = 10000
E = 160000
D = 256
EPS = 1e-5

NC = 2
NS = 16
NW = NC * NS

OWN = 320
NPADR = NW * OWN
DUMP = OWN

STRIP = 2000
NSTRIP = E // STRIP
SELCAP = 2048
CH = 64


def _mesh():
    return plsc.VectorSubcoreMesh(core_axis_name="c", subcore_axis_name="s")


_SC_PARAMS = pltpu.CompilerParams(needs_layout_passes=False)


def _wid():
    return lax.axis_index("c") * NS + lax.axis_index("s")


def _deg_body(dst_hbm, out_hbm, dstst, selloc, deg2d, sem):
    w = _wid()
    lo = w * OWN
    one16 = jnp.ones((16,), jnp.float32)
    dump16 = jnp.full((16,), DUMP, jnp.int32)
    iota16 = lax.iota(jnp.int32, 16)

    def init(i, c):
        deg2d[i, pl.ds(0, 16)] = one16
        return c
    lax.fori_loop(0, OWN // 16 + 1, init, 0)

    def strip(s, c):
        pltpu.sync_copy(dst_hbm.at[pl.ds(s * STRIP, STRIP)], dstst)

        def pf(i, c2):
            selloc[pl.ds(i * 16, 16)] = dump16
            return c2
        lax.fori_loop(0, SELCAP // 16, pf, 0)

        def grp(g, off):
            v = dstst[pl.ds(g * 16, 16)]
            msk = (v >= lo) & (v < lo + OWN)
            pm = plsc.cumsum(msk.astype(jnp.int32))
            pos = off + pm - 1
            plsc.store_scatter(selloc, [pos], v - lo, mask=msk)
            return off + pm[15]
        m = lax.fori_loop(0, STRIP // 16, grp, jnp.int32(0))

        def inc(g, c2):
            dlv = selloc[pl.ds(g * 16, 16)]
            for t in range(16):
                dl = jnp.minimum(jnp.maximum(dlv[t], 0), DUMP)
                row = dl // 16
                onehot = jnp.where(iota16 == dl - row * 16, 1.0, 0.0)
                plsc.addupdate(deg2d.at[row], onehot)
            return c2
        lax.fori_loop(0, (m + 15) // 16, inc, 0)
        return c
    lax.fori_loop(0, NSTRIP, strip, 0)

    pltpu.sync_copy(deg2d.at[pl.ds(0, OWN // 16)], out_hbm.at[w])


def _deg_call(dst):
    out = pl.kernel(
        _deg_body,
        out_type=jax.ShapeDtypeStruct((NW, OWN // 16, 16), jnp.float32),
        mesh=_mesh(),
        scratch_types=[
            pltpu.VMEM((STRIP,), jnp.int32),
            pltpu.VMEM((SELCAP,), jnp.int32),
            pltpu.VMEM((OWN // 16 + 1, 16), jnp.float32),
            pltpu.SemaphoreType.DMA,
        ],
        compiler_params=_SC_PARAMS,
    )(dst)
    return out.reshape(NPADR)


def _agg_body(hp_hbm, src_hbm, dst_hbm, out_hbm,
              dstst, srcst, selbuf, selloc, gbuf, acc, sem):
    w = _wid()
    lo = w * OWN
    zero16 = jnp.zeros((16,), jnp.int32)
    dump16 = jnp.full((16,), DUMP, jnp.int32)

    pltpu.sync_copy(hp_hbm.at[pl.ds(lo, OWN)], acc.at[pl.ds(0, OWN)])

    def strip(s, c):
        pltpu.sync_copy(dst_hbm.at[pl.ds(s * STRIP, STRIP)], dstst)
        pltpu.sync_copy(src_hbm.at[pl.ds(s * STRIP, STRIP)], srcst)

        def pf(i, c2):
            selbuf[pl.ds(i * 16, 16)] = zero16
            selloc[pl.ds(i * 16, 16)] = dump16
            return c2
        lax.fori_loop(0, SELCAP // 16, pf, 0)

        def grp(g, off):
            v = dstst[pl.ds(g * 16, 16)]
            sv = srcst[pl.ds(g * 16, 16)]
            msk = (v >= lo) & (v < lo + OWN)
            pm = plsc.cumsum(msk.astype(jnp.int32))
            pos = off + pm - 1
            plsc.store_scatter(selbuf, [pos], sv, mask=msk)
            plsc.store_scatter(selloc, [pos], v - lo, mask=msk)
            return off + pm[15]
        m = lax.fori_loop(0, STRIP // 16, grp, jnp.int32(0))

        nch = (m + (CH - 1)) // CH

        iota16 = lax.iota(jnp.int32, 16)
        perms = [(iota16 + sh) % 16 for sh in range(16)]

        def chunk(c2, carry):
            pltpu.async_copy(hp_hbm.at[selbuf.at[pl.ds(c2 * CH, CH)]],
                             gbuf, sem).wait()

            def egroup(g, c3):
                dlv = jnp.clip(selloc[pl.ds(c2 * CH + g * 16, 16)], 0, DUMP)
                jvec = g * 16 + iota16

                def cblk(cb, c4):
                    for sh in range(16):
                        colv = cb * 16 + perms[sh]
                        vals = plsc.load_gather(gbuf, [jvec, colv])
                        plsc.addupdate_scatter(acc, [dlv, colv], vals)
                    return c4
                lax.fori_loop(0, D // 16, cblk, 0)
                return c3
            lax.fori_loop(0, CH // 16, egroup, 0)
            return carry
        lax.fori_loop(0, nch, chunk, 0)
        return c
    lax.fori_loop(0, NSTRIP, strip, 0)

    pltpu.sync_copy(acc.at[pl.ds(0, OWN)], out_hbm.at[pl.ds(lo, OWN)])


def _agg_call(hp_pad, src, dst):
    return pl.kernel(
        _agg_body,
        out_type=jax.ShapeDtypeStruct((NPADR, D), jnp.float32),
        mesh=_mesh(),
        scratch_types=[
            pltpu.VMEM((STRIP,), jnp.int32),
            pltpu.VMEM((STRIP,), jnp.int32),
            pltpu.VMEM((SELCAP,), jnp.int32),
            pltpu.VMEM((SELCAP,), jnp.int32),
            pltpu.VMEM((CH, D), jnp.float32),
            pltpu.VMEM((OWN + 8, D), jnp.float32),
            pltpu.SemaphoreType.DMA,
        ],
        compiler_params=_SC_PARAMS,
    )(hp_pad, src, dst)


RB = 1000
NRB = N // RB


def _dinv_block(deg_ref):
    return lax.rsqrt(deg_ref[...])


def _t0_body(x_ref, w_ref, deg_ref, out_ref):
    out_ref[...] = jnp.dot(x_ref[...], w_ref[...],
                           preferred_element_type=jnp.float32) * _dinv_block(deg_ref)


def _t0_call(x, W1, deg):
    return pl.pallas_call(
        _t0_body,
        grid=(NRB,),
        in_specs=[
            pl.BlockSpec((RB, D), lambda r: (r, 0)),
            pl.BlockSpec((D, D), lambda r: (0, 0)),
            pl.BlockSpec((RB, 1), lambda r: (r, 0)),
        ],
        out_specs=pl.BlockSpec((RB, D), lambda r: (r, 0)),
        out_shape=jax.ShapeDtypeStruct((N, D), jnp.float32),
    )(x, W1, deg)


def _stats_body(agg_ref, deg_ref, c_out, st_out, acc):
    r = pl.program_id(0)

    @pl.when(r == 0)
    def _():
        acc[...] = jnp.zeros_like(acc)

    cb = _dinv_block(deg_ref) * agg_ref[...]
    c_out[...] = cb
    acc[0:1, :] += jnp.sum(cb, axis=0, keepdims=True)
    acc[1:2, :] += jnp.sum(cb * cb, axis=0, keepdims=True)

    @pl.when(r == NRB - 1)
    def _():
        mean = acc[0:1, :] * (1.0 / N)
        var = acc[1:2, :] * (1.0 / N) - mean * mean
        st_out[0:1, :] = mean
        st_out[1:2, :] = lax.rsqrt(var + EPS)


def _stats_call(agg, deg):
    return pl.pallas_call(
        _stats_body,
        grid=(NRB,),
        in_specs=[
            pl.BlockSpec((RB, D), lambda r: (r, 0)),
            pl.BlockSpec((RB, 1), lambda r: (r, 0)),
        ],
        out_specs=[
            pl.BlockSpec((RB, D), lambda r: (r, 0)),
            pl.BlockSpec((2, D), lambda r: (0, 0)),
        ],
        out_shape=[
            jax.ShapeDtypeStruct((N, D), jnp.float32),
            jax.ShapeDtypeStruct((2, D), jnp.float32),
        ],
        scratch_shapes=[pltpu.VMEM((2, D), jnp.float32)],
    )(agg, deg)


def _t1b_body(c_ref, st_ref, g_ref, b_ref, w_ref, deg_ref, out_ref):
    y = jnp.maximum(
        g_ref[...] * (c_ref[...] - st_ref[0:1, :]) * st_ref[1:2, :] + b_ref[...],
        0.0)
    out_ref[...] = jnp.dot(y, w_ref[...],
                           preferred_element_type=jnp.float32) * _dinv_block(deg_ref)


def _t1b_call(c1, st1, g, b, W2, deg):
    return pl.pallas_call(
        _t1b_body,
        grid=(NRB,),
        in_specs=[
            pl.BlockSpec((RB, D), lambda r: (r, 0)),
            pl.BlockSpec((2, D), lambda r: (0, 0)),
            pl.BlockSpec((1, D), lambda r: (0, 0)),
            pl.BlockSpec((1, D), lambda r: (0, 0)),
            pl.BlockSpec((D, D), lambda r: (0, 0)),
            pl.BlockSpec((RB, 1), lambda r: (r, 0)),
        ],
        out_specs=pl.BlockSpec((RB, D), lambda r: (r, 0)),
        out_shape=jax.ShapeDtypeStruct((N, D), jnp.float32),
    )(c1, st1, g, b, W2, deg)


def _t2b_body(c_ref, st_ref, g_ref, b_ref, x_ref, out_ref):
    out_ref[...] = jnp.maximum(
        g_ref[...] * (c_ref[...] - st_ref[0:1, :]) * st_ref[1:2, :]
        + b_ref[...] + x_ref[...],
        0.0)


def _t2b_call(c2, st2, g, b, x):
    return pl.pallas_call(
        _t2b_body,
        grid=(NRB,),
        in_specs=[
            pl.BlockSpec((RB, D), lambda r: (r, 0)),
            pl.BlockSpec((2, D), lambda r: (0, 0)),
            pl.BlockSpec((1, D), lambda r: (0, 0)),
            pl.BlockSpec((1, D), lambda r: (0, 0)),
            pl.BlockSpec((RB, D), lambda r: (r, 0)),
        ],
        out_specs=pl.BlockSpec((RB, D), lambda r: (r, 0)),
        out_shape=jax.ShapeDtypeStruct((N, D), jnp.float32),
    )(c2, st2, g, b, x)


def kernel(x, edge_index, W1, b1, bn1_w, bn1_b, W2, b2, bn2_w, bn2_b):
    src = edge_index[0]
    dst = edge_index[1]
    deg = _deg_call(dst).reshape(NPADR, 1)
    h1p = _t0_call(x, W1, deg)
    agg1 = _agg_call(jnp.pad(h1p, ((0, NPADR - N), (0, 0))), src, dst)
    c1, st1 = _stats_call(agg1, deg)
    h2p = _t1b_call(c1, st1, bn1_w.reshape(1, D), bn1_b.reshape(1, D), W2, deg)
    agg2 = _agg_call(jnp.pad(h2p, ((0, NPADR - N), (0, 0))), src, dst)
    c2, st2 = _stats_call(agg2, deg)
    return _t2b_call(c2, st2, bn2_w.reshape(1, D), bn2_b.reshape(1, D), x)


# speedup vs baseline: 2.9875x; 2.9875x over previous
import jax
import jax.numpy as jnp
from jax import lax
from jax.experimental import pallas as pl
from jax.experimental.pallas import tpu as pltpu
from jax.experimental.pallas import tpu_sc as plsc

N = 10000
E = 160000
D = 256
EPS = 1e-5

NC = 2
NS = 16
NW = NC * NS

OWN = 320
NPADR = NW * OWN
DUMP = OWN

STRIP = 2000
NSTRIP = E // STRIP
SELCAP = 2048
CH = 64


def _mesh():
    return plsc.VectorSubcoreMesh(core_axis_name="c", subcore_axis_name="s")


_SC_PARAMS = pltpu.CompilerParams(needs_layout_passes=False)


def _wid():
    return lax.axis_index("c") * NS + lax.axis_index("s")


def _deg_body(dst_hbm, out_hbm, dstst, selloc, deg2d, sem):
    w = _wid()
    lo = w * OWN
    one16 = jnp.ones((16,), jnp.float32)
    dump16 = jnp.full((16,), DUMP, jnp.int32)
    iota16 = lax.iota(jnp.int32, 16)

    def init(i, c):
        deg2d[i, pl.ds(0, 16)] = one16
        return c
    lax.fori_loop(0, OWN // 16 + 1, init, 0)

    def strip(s, c):
        pltpu.sync_copy(dst_hbm.at[pl.ds(s * STRIP, STRIP)], dstst)

        def pf(i, c2):
            selloc[pl.ds(i * 16, 16)] = dump16
            return c2
        lax.fori_loop(0, SELCAP // 16, pf, 0)

        def grp(g, off):
            v = dstst[pl.ds(g * 16, 16)]
            msk = (v >= lo) & (v < lo + OWN)
            pm = plsc.cumsum(msk.astype(jnp.int32))
            pos = off + pm - 1
            plsc.store_scatter(selloc, [pos], v - lo, mask=msk)
            return off + pm[15]
        m = lax.fori_loop(0, STRIP // 16, grp, jnp.int32(0))

        def inc(g, c2):
            dlv = selloc[pl.ds(g * 16, 16)]
            for t in range(16):
                dl = jnp.minimum(jnp.maximum(dlv[t], 0), DUMP)
                row = dl // 16
                onehot = jnp.where(iota16 == dl - row * 16, 1.0, 0.0)
                plsc.addupdate(deg2d.at[row], onehot)
            return c2
        lax.fori_loop(0, (m + 15) // 16, inc, 0)
        return c
    lax.fori_loop(0, NSTRIP, strip, 0)

    pltpu.sync_copy(deg2d.at[pl.ds(0, OWN // 16)], out_hbm.at[w])


def _deg_call(dst):
    out = pl.kernel(
        _deg_body,
        out_type=jax.ShapeDtypeStruct((NW, OWN // 16, 16), jnp.float32),
        mesh=_mesh(),
        scratch_types=[
            pltpu.VMEM((STRIP,), jnp.int32),
            pltpu.VMEM((SELCAP,), jnp.int32),
            pltpu.VMEM((OWN // 16 + 1, 16), jnp.float32),
            pltpu.SemaphoreType.DMA,
        ],
        compiler_params=_SC_PARAMS,
    )(dst)
    return out.reshape(NPADR)


def _agg_body(hp_hbm, src_hbm, dst_hbm, out_hbm,
              dstst, srcst, selbuf, selloc, gbuf, acc, sem):
    w = _wid()
    lo = w * OWN
    zero16 = jnp.zeros((16,), jnp.int32)
    dump16 = jnp.full((16,), DUMP, jnp.int32)

    pltpu.sync_copy(hp_hbm.at[pl.ds(lo, OWN)], acc.at[pl.ds(0, OWN)])

    iota16 = lax.iota(jnp.int32, 16)
    perms = [(iota16 + sh) % 16 for sh in range(16)]

    def strip(s, c):
        pltpu.sync_copy(dst_hbm.at[pl.ds(s * STRIP, STRIP)], dstst)
        pltpu.sync_copy(src_hbm.at[pl.ds(s * STRIP, STRIP)], srcst)

        def pf(i, c2):
            selbuf[pl.ds(i * 16, 16)] = i * 16 + iota16
            selloc[pl.ds(i * 16, 16)] = dump16
            return c2
        lax.fori_loop(0, SELCAP // 16, pf, 0)

        def grp(g, off):
            v = dstst[pl.ds(g * 16, 16)]
            sv = srcst[pl.ds(g * 16, 16)]
            msk = (v >= lo) & (v < lo + OWN)
            pm = plsc.cumsum(msk.astype(jnp.int32))
            pos = off + pm - 1
            plsc.store_scatter(selbuf, [pos], sv, mask=msk)
            plsc.store_scatter(selloc, [pos], v - lo, mask=msk)
            return off + pm[15]
        m = lax.fori_loop(0, STRIP // 16, grp, jnp.int32(0))

        nch = (m + (CH - 1)) // CH

        def do_chunk(c2):
            pltpu.async_copy(hp_hbm.at[selbuf.at[pl.ds(c2 * CH, CH)]],
                             gbuf, sem).wait()

            def egroup(g, c3):
                dlv = jnp.clip(selloc[pl.ds(c2 * CH + g * 16, 16)], 0, DUMP)
                jvec = g * 16 + iota16

                def cblk(cb, c4):
                    for sh in range(16):
                        colv = cb * 16 + perms[sh]
                        vals = plsc.load_gather(gbuf, [jvec, colv])
                        plsc.addupdate_scatter(acc, [dlv, colv], vals)
                    return c4
                lax.fori_loop(0, D // 16, cblk, 0)
                return c3
            lax.fori_loop(0, CH // 16, egroup, 0)

        do_chunk(0)

        @pl.when(m > CH)
        def _():
            do_chunk(1)

        def chunk(c2, carry):
            do_chunk(c2)
            return carry
        lax.fori_loop(2, nch, chunk, 0)
        return c
    lax.fori_loop(0, NSTRIP, strip, 0)

    pltpu.sync_copy(acc.at[pl.ds(0, OWN)], out_hbm.at[pl.ds(lo, OWN)])


def _agg_call(hp_pad, src, dst):
    return pl.kernel(
        _agg_body,
        out_type=jax.ShapeDtypeStruct((NPADR, D), jnp.float32),
        mesh=_mesh(),
        scratch_types=[
            pltpu.VMEM((STRIP,), jnp.int32),
            pltpu.VMEM((STRIP,), jnp.int32),
            pltpu.VMEM((SELCAP,), jnp.int32),
            pltpu.VMEM((SELCAP,), jnp.int32),
            pltpu.VMEM((CH, D), jnp.float32),
            pltpu.VMEM((OWN + 8, D), jnp.float32),
            pltpu.SemaphoreType.DMA,
        ],
        compiler_params=_SC_PARAMS,
    )(hp_pad, src, dst)


RB = 1000
NRB = N // RB


def _dinv_block(deg_ref):
    return lax.rsqrt(deg_ref[...])


def _t0_body(x_ref, w_ref, deg_ref, out_ref):
    out_ref[...] = jnp.dot(x_ref[...], w_ref[...],
                           preferred_element_type=jnp.float32) * _dinv_block(deg_ref)


def _t0_call(x, W1, deg):
    return pl.pallas_call(
        _t0_body,
        grid=(NRB,),
        in_specs=[
            pl.BlockSpec((RB, D), lambda r: (r, 0)),
            pl.BlockSpec((D, D), lambda r: (0, 0)),
            pl.BlockSpec((RB, 1), lambda r: (r, 0)),
        ],
        out_specs=pl.BlockSpec((RB, D), lambda r: (r, 0)),
        out_shape=jax.ShapeDtypeStruct((N, D), jnp.float32),
    )(x, W1, deg)


def _stats_body(agg_ref, deg_ref, c_out, st_out, acc):
    r = pl.program_id(0)

    @pl.when(r == 0)
    def _():
        acc[...] = jnp.zeros_like(acc)

    cb = _dinv_block(deg_ref) * agg_ref[...]
    c_out[...] = cb
    acc[0:1, :] += jnp.sum(cb, axis=0, keepdims=True)
    acc[1:2, :] += jnp.sum(cb * cb, axis=0, keepdims=True)

    @pl.when(r == NRB - 1)
    def _():
        mean = acc[0:1, :] * (1.0 / N)
        var = acc[1:2, :] * (1.0 / N) - mean * mean
        st_out[0:1, :] = mean
        st_out[1:2, :] = lax.rsqrt(var + EPS)


def _stats_call(agg, deg):
    return pl.pallas_call(
        _stats_body,
        grid=(NRB,),
        in_specs=[
            pl.BlockSpec((RB, D), lambda r: (r, 0)),
            pl.BlockSpec((RB, 1), lambda r: (r, 0)),
        ],
        out_specs=[
            pl.BlockSpec((RB, D), lambda r: (r, 0)),
            pl.BlockSpec((2, D), lambda r: (0, 0)),
        ],
        out_shape=[
            jax.ShapeDtypeStruct((N, D), jnp.float32),
            jax.ShapeDtypeStruct((2, D), jnp.float32),
        ],
        scratch_shapes=[pltpu.VMEM((2, D), jnp.float32)],
    )(agg, deg)


def _t1b_body(c_ref, st_ref, g_ref, b_ref, w_ref, deg_ref, out_ref):
    y = jnp.maximum(
        g_ref[...] * (c_ref[...] - st_ref[0:1, :]) * st_ref[1:2, :] + b_ref[...],
        0.0)
    out_ref[...] = jnp.dot(y, w_ref[...],
                           preferred_element_type=jnp.float32) * _dinv_block(deg_ref)


def _t1b_call(c1, st1, g, b, W2, deg):
    return pl.pallas_call(
        _t1b_body,
        grid=(NRB,),
        in_specs=[
            pl.BlockSpec((RB, D), lambda r: (r, 0)),
            pl.BlockSpec((2, D), lambda r: (0, 0)),
            pl.BlockSpec((1, D), lambda r: (0, 0)),
            pl.BlockSpec((1, D), lambda r: (0, 0)),
            pl.BlockSpec((D, D), lambda r: (0, 0)),
            pl.BlockSpec((RB, 1), lambda r: (r, 0)),
        ],
        out_specs=pl.BlockSpec((RB, D), lambda r: (r, 0)),
        out_shape=jax.ShapeDtypeStruct((N, D), jnp.float32),
    )(c1, st1, g, b, W2, deg)


def _t2b_body(c_ref, st_ref, g_ref, b_ref, x_ref, out_ref):
    out_ref[...] = jnp.maximum(
        g_ref[...] * (c_ref[...] - st_ref[0:1, :]) * st_ref[1:2, :]
        + b_ref[...] + x_ref[...],
        0.0)


def _t2b_call(c2, st2, g, b, x):
    return pl.pallas_call(
        _t2b_body,
        grid=(NRB,),
        in_specs=[
            pl.BlockSpec((RB, D), lambda r: (r, 0)),
            pl.BlockSpec((2, D), lambda r: (0, 0)),
            pl.BlockSpec((1, D), lambda r: (0, 0)),
            pl.BlockSpec((1, D), lambda r: (0, 0)),
            pl.BlockSpec((RB, D), lambda r: (r, 0)),
        ],
        out_specs=pl.BlockSpec((RB, D), lambda r: (r, 0)),
        out_shape=jax.ShapeDtypeStruct((N, D), jnp.float32),
    )(c2, st2, g, b, x)


def kernel(x, edge_index, W1, b1, bn1_w, bn1_b, W2, b2, bn2_w, bn2_b):
    src = edge_index[0]
    dst = edge_index[1]
    deg = _deg_call(dst).reshape(NPADR, 1)
    h1p = _t0_call(x, W1, deg)
    agg1 = _agg_call(jnp.pad(h1p, ((0, NPADR - N), (0, 0))), src, dst)
    c1, st1 = _stats_call(agg1, deg)
    h2p = _t1b_call(c1, st1, bn1_w.reshape(1, D), bn1_b.reshape(1, D), W2, deg)
    agg2 = _agg_call(jnp.pad(h2p, ((0, NPADR - N), (0, 0))), src, dst)
    c2, st2 = _stats_call(agg2, deg)
    return _t2b_call(c2, st2, bn2_w.reshape(1, D), bn2_b.reshape(1, D), x)


# speedup vs baseline: 3.3668x; 1.1270x over previous
import jax
import jax.numpy as jnp
from jax import lax
from jax.experimental import pallas as pl
from jax.experimental.pallas import tpu as pltpu
from jax.experimental.pallas import tpu_sc as plsc

N = 10000
E = 160000
D = 256
EPS = 1e-5

NC = 2
NS = 16
NW = NC * NS

OWN = 320
NPADR = NW * OWN
DUMP = OWN

STRIP = 2000
NSTRIP = E // STRIP
SELCAP = 2048
CH = 64


def _mesh():
    return plsc.VectorSubcoreMesh(core_axis_name="c", subcore_axis_name="s")


_SC_PARAMS = pltpu.CompilerParams(needs_layout_passes=False)


def _wid():
    return lax.axis_index("c") * NS + lax.axis_index("s")


def _prep_body(src_hbm, dst_hbm, deg_out, sb_out, sl_out, cnt_out,
               dstst, srcst, selbuf, selloc, deg2d, cntv, sem):
    w = _wid()
    lo = w * OWN
    one16 = jnp.ones((16,), jnp.float32)
    dump16 = jnp.full((16,), DUMP, jnp.int32)
    iota16 = lax.iota(jnp.int32, 16)

    def init(i, c):
        deg2d[i, pl.ds(0, 16)] = one16
        return c
    lax.fori_loop(0, OWN // 16 + 1, init, 0)

    def strip(s, c):
        pltpu.sync_copy(dst_hbm.at[pl.ds(s * STRIP, STRIP)], dstst)
        pltpu.sync_copy(src_hbm.at[pl.ds(s * STRIP, STRIP)], srcst)

        def pf(i, c2):
            selbuf[pl.ds(i * 16, 16)] = i * 16 + iota16
            selloc[pl.ds(i * 16, 16)] = dump16
            return c2
        lax.fori_loop(0, SELCAP // 16, pf, 0)

        def grp(g, off):
            v = dstst[pl.ds(g * 16, 16)]
            sv = srcst[pl.ds(g * 16, 16)]
            msk = (v >= lo) & (v < lo + OWN)
            pm = plsc.cumsum(msk.astype(jnp.int32))
            pos = off + pm - 1
            plsc.store_scatter(selbuf, [pos], sv, mask=msk)
            plsc.store_scatter(selloc, [pos], v - lo, mask=msk)
            return off + pm[15]
        m = lax.fori_loop(0, STRIP // 16, grp, jnp.int32(0))

        cntv[pl.ds(s * 16, 16)] = jnp.broadcast_to(m, (16,)).astype(jnp.int32)

        def inc(g, c2):
            dlv = selloc[pl.ds(g * 16, 16)]
            for t in range(16):
                dl = jnp.minimum(jnp.maximum(dlv[t], 0), DUMP)
                row = dl // 16
                onehot = jnp.where(iota16 == dl - row * 16, 1.0, 0.0)
                plsc.addupdate(deg2d.at[row], onehot)
            return c2
        lax.fori_loop(0, (m + 15) // 16, inc, 0)

        pltpu.sync_copy(selbuf, sb_out.at[w, s])
        pltpu.sync_copy(selloc, sl_out.at[w, s])
        return c
    lax.fori_loop(0, NSTRIP, strip, 0)

    pltpu.sync_copy(deg2d.at[pl.ds(0, OWN // 16)], deg_out.at[w])
    pltpu.sync_copy(cntv, cnt_out.at[w])


def _prep_call(src, dst):
    deg, sb, sl, cnt = pl.kernel(
        _prep_body,
        out_type=(
            jax.ShapeDtypeStruct((NW, OWN // 16, 16), jnp.float32),
            jax.ShapeDtypeStruct((NW, NSTRIP, SELCAP), jnp.int32),
            jax.ShapeDtypeStruct((NW, NSTRIP, SELCAP), jnp.int32),
            jax.ShapeDtypeStruct((NW, NSTRIP * 16), jnp.int32),
        ),
        mesh=_mesh(),
        scratch_types=[
            pltpu.VMEM((STRIP,), jnp.int32),
            pltpu.VMEM((STRIP,), jnp.int32),
            pltpu.VMEM((SELCAP,), jnp.int32),
            pltpu.VMEM((SELCAP,), jnp.int32),
            pltpu.VMEM((OWN // 16 + 1, 16), jnp.float32),
            pltpu.VMEM((NSTRIP * 16,), jnp.int32),
            pltpu.SemaphoreType.DMA,
        ],
        compiler_params=_SC_PARAMS,
    )(src, dst)
    return deg.reshape(NPADR), sb, sl, cnt


def _agg_body(hp_hbm, sb_hbm, sl_hbm, cnt_hbm, out_hbm,
              selbuf, selloc, cntv, gbuf, acc, sem):
    w = _wid()
    lo = w * OWN

    pltpu.sync_copy(hp_hbm.at[pl.ds(lo, OWN)], acc.at[pl.ds(0, OWN)])
    pltpu.sync_copy(cnt_hbm.at[w], cntv)

    iota16 = lax.iota(jnp.int32, 16)
    perms = [(iota16 + sh) % 16 for sh in range(16)]

    def strip(s, c):
        pltpu.sync_copy(sb_hbm.at[w, s], selbuf)
        pltpu.sync_copy(sl_hbm.at[w, s], selloc)
        m = cntv[pl.ds(s * 16, 16)][0]
        nch = (m + (CH - 1)) // CH

        def chunk(c2, carry):
            pltpu.async_copy(hp_hbm.at[selbuf.at[pl.ds(c2 * CH, CH)]],
                             gbuf, sem).wait()

            def egroup(g, c3):
                dlv = jnp.clip(selloc[pl.ds(c2 * CH + g * 16, 16)], 0, DUMP)
                jvec = g * 16 + iota16

                def cblk(cb, c4):
                    for sh in range(16):
                        colv = cb * 16 + perms[sh]
                        vals = plsc.load_gather(gbuf, [jvec, colv])
                        plsc.addupdate_scatter(acc, [dlv, colv], vals)
                    return c4
                lax.fori_loop(0, D // 16, cblk, 0)
                return c3
            lax.fori_loop(0, CH // 16, egroup, 0)
            return carry
        lax.fori_loop(0, nch, chunk, 0)
        return c
    lax.fori_loop(0, NSTRIP, strip, 0)

    pltpu.sync_copy(acc.at[pl.ds(0, OWN)], out_hbm.at[pl.ds(lo, OWN)])


def _agg_call(hp_pad, sb, sl, cnt):
    return pl.kernel(
        _agg_body,
        out_type=jax.ShapeDtypeStruct((NPADR, D), jnp.float32),
        mesh=_mesh(),
        scratch_types=[
            pltpu.VMEM((SELCAP,), jnp.int32),
            pltpu.VMEM((SELCAP,), jnp.int32),
            pltpu.VMEM((NSTRIP * 16,), jnp.int32),
            pltpu.VMEM((CH, D), jnp.float32),
            pltpu.VMEM((OWN + 8, D), jnp.float32),
            pltpu.SemaphoreType.DMA,
        ],
        compiler_params=_SC_PARAMS,
    )(hp_pad, sb, sl, cnt)


RB = 1000
NRB = N // RB


def _dinv_block(deg_ref):
    return lax.rsqrt(deg_ref[...])


def _t0_body(x_ref, w_ref, deg_ref, out_ref):
    out_ref[...] = jnp.dot(x_ref[...], w_ref[...],
                           preferred_element_type=jnp.float32) * _dinv_block(deg_ref)


def _t0_call(x, W1, deg):
    return pl.pallas_call(
        _t0_body,
        grid=(NRB,),
        in_specs=[
            pl.BlockSpec((RB, D), lambda r: (r, 0)),
            pl.BlockSpec((D, D), lambda r: (0, 0)),
            pl.BlockSpec((RB, 1), lambda r: (r, 0)),
        ],
        out_specs=pl.BlockSpec((RB, D), lambda r: (r, 0)),
        out_shape=jax.ShapeDtypeStruct((N, D), jnp.float32),
    )(x, W1, deg)


def _stats_body(agg_ref, deg_ref, c_out, st_out, acc):
    r = pl.program_id(0)

    @pl.when(r == 0)
    def _():
        acc[...] = jnp.zeros_like(acc)

    cb = _dinv_block(deg_ref) * agg_ref[...]
    c_out[...] = cb
    acc[0:1, :] += jnp.sum(cb, axis=0, keepdims=True)
    acc[1:2, :] += jnp.sum(cb * cb, axis=0, keepdims=True)

    @pl.when(r == NRB - 1)
    def _():
        mean = acc[0:1, :] * (1.0 / N)
        var = acc[1:2, :] * (1.0 / N) - mean * mean
        st_out[0:1, :] = mean
        st_out[1:2, :] = lax.rsqrt(var + EPS)


def _stats_call(agg, deg):
    return pl.pallas_call(
        _stats_body,
        grid=(NRB,),
        in_specs=[
            pl.BlockSpec((RB, D), lambda r: (r, 0)),
            pl.BlockSpec((RB, 1), lambda r: (r, 0)),
        ],
        out_specs=[
            pl.BlockSpec((RB, D), lambda r: (r, 0)),
            pl.BlockSpec((2, D), lambda r: (0, 0)),
        ],
        out_shape=[
            jax.ShapeDtypeStruct((N, D), jnp.float32),
            jax.ShapeDtypeStruct((2, D), jnp.float32),
        ],
        scratch_shapes=[pltpu.VMEM((2, D), jnp.float32)],
    )(agg, deg)


def _t1b_body(c_ref, st_ref, g_ref, b_ref, w_ref, deg_ref, out_ref):
    y = jnp.maximum(
        g_ref[...] * (c_ref[...] - st_ref[0:1, :]) * st_ref[1:2, :] + b_ref[...],
        0.0)
    out_ref[...] = jnp.dot(y, w_ref[...],
                           preferred_element_type=jnp.float32) * _dinv_block(deg_ref)


def _t1b_call(c1, st1, g, b, W2, deg):
    return pl.pallas_call(
        _t1b_body,
        grid=(NRB,),
        in_specs=[
            pl.BlockSpec((RB, D), lambda r: (r, 0)),
            pl.BlockSpec((2, D), lambda r: (0, 0)),
            pl.BlockSpec((1, D), lambda r: (0, 0)),
            pl.BlockSpec((1, D), lambda r: (0, 0)),
            pl.BlockSpec((D, D), lambda r: (0, 0)),
            pl.BlockSpec((RB, 1), lambda r: (r, 0)),
        ],
        out_specs=pl.BlockSpec((RB, D), lambda r: (r, 0)),
        out_shape=jax.ShapeDtypeStruct((N, D), jnp.float32),
    )(c1, st1, g, b, W2, deg)


def _t2b_body(c_ref, st_ref, g_ref, b_ref, x_ref, out_ref):
    out_ref[...] = jnp.maximum(
        g_ref[...] * (c_ref[...] - st_ref[0:1, :]) * st_ref[1:2, :]
        + b_ref[...] + x_ref[...],
        0.0)


def _t2b_call(c2, st2, g, b, x):
    return pl.pallas_call(
        _t2b_body,
        grid=(NRB,),
        in_specs=[
            pl.BlockSpec((RB, D), lambda r: (r, 0)),
            pl.BlockSpec((2, D), lambda r: (0, 0)),
            pl.BlockSpec((1, D), lambda r: (0, 0)),
            pl.BlockSpec((1, D), lambda r: (0, 0)),
            pl.BlockSpec((RB, D), lambda r: (r, 0)),
        ],
        out_specs=pl.BlockSpec((RB, D), lambda r: (r, 0)),
        out_shape=jax.ShapeDtypeStruct((N, D), jnp.float32),
    )(c2, st2, g, b, x)


def kernel(x, edge_index, W1, b1, bn1_w, bn1_b, W2, b2, bn2_w, bn2_b):
    src = edge_index[0]
    dst = edge_index[1]
    deg, sb, sl, cnt = _prep_call(src, dst)
    deg = deg.reshape(NPADR, 1)
    h1p = _t0_call(x, W1, deg)
    agg1 = _agg_call(jnp.pad(h1p, ((0, NPADR - N), (0, 0))), sb, sl, cnt)
    c1, st1 = _stats_call(agg1, deg)
    h2p = _t1b_call(c1, st1, bn1_w.reshape(1, D), bn1_b.reshape(1, D), W2, deg)
    agg2 = _agg_call(jnp.pad(h2p, ((0, NPADR - N), (0, 0))), sb, sl, cnt)
    c2, st2 = _stats_call(agg2, deg)
    return _t2b_call(c2, st2, bn2_w.reshape(1, D), bn2_b.reshape(1, D), x)


# speedup vs baseline: 3.7107x; 1.1022x over previous
import jax
import jax.numpy as jnp
from jax import lax
from jax.experimental import pallas as pl
from jax.experimental.pallas import tpu as pltpu
from jax.experimental.pallas import tpu_sc as plsc

N = 10000
E = 160000
D = 256
EPS = 1e-5

NC = 2
NS = 16
NW = NC * NS

OWN = 320
NPADR = NW * OWN
DUMP = OWN

STRIP = 2000
NSTRIP = E // STRIP
SELCAP = 2048
CH = 64


def _mesh():
    return plsc.VectorSubcoreMesh(core_axis_name="c", subcore_axis_name="s")


_SC_PARAMS = pltpu.CompilerParams(needs_layout_passes=False)


def _wid():
    return lax.axis_index("c") * NS + lax.axis_index("s")


def _prep_body(src_hbm, dst_hbm, deg_out, sbsl_out, cnt_out,
               dstst, srcst, selbuf, selloc, deg2d, cntv, sem):
    w = _wid()
    lo = w * OWN
    one16 = jnp.ones((16,), jnp.float32)
    dump16 = jnp.full((16,), DUMP, jnp.int32)
    iota16 = lax.iota(jnp.int32, 16)

    def init(i, c):
        deg2d[i, pl.ds(0, 16)] = one16
        return c
    lax.fori_loop(0, OWN // 16 + 1, init, 0)

    def strip(s, c):
        pltpu.sync_copy(dst_hbm.at[pl.ds(s * STRIP, STRIP)], dstst)
        pltpu.sync_copy(src_hbm.at[pl.ds(s * STRIP, STRIP)], srcst)

        def pf(i, c2):
            selbuf[pl.ds(i * 16, 16)] = i * 16 + iota16
            selloc[pl.ds(i * 16, 16)] = dump16
            return c2
        lax.fori_loop(0, SELCAP // 16, pf, 0)

        def grp(g, off):
            v = dstst[pl.ds(g * 16, 16)]
            sv = srcst[pl.ds(g * 16, 16)]
            msk = (v >= lo) & (v < lo + OWN)
            pm = plsc.cumsum(msk.astype(jnp.int32))
            pos = off + pm - 1
            plsc.store_scatter(selbuf, [pos], sv, mask=msk)
            plsc.store_scatter(selloc, [pos], v - lo, mask=msk)
            return off + pm[15]
        m = lax.fori_loop(0, STRIP // 16, grp, jnp.int32(0))

        cntv[pl.ds(s * 16, 16)] = jnp.broadcast_to(m, (16,)).astype(jnp.int32)

        def inc(g, c2):
            dlv = selloc[pl.ds(g * 16, 16)]
            for t in range(16):
                dl = jnp.minimum(jnp.maximum(dlv[t], 0), DUMP)
                row = dl // 16
                onehot = jnp.where(iota16 == dl - row * 16, 1.0, 0.0)
                plsc.addupdate(deg2d.at[row], onehot)
            return c2
        lax.fori_loop(0, (m + 15) // 16, inc, 0)

        pltpu.sync_copy(selbuf, sbsl_out.at[w, s, 0])
        pltpu.sync_copy(selloc, sbsl_out.at[w, s, 1])
        return c
    lax.fori_loop(0, NSTRIP, strip, 0)

    pltpu.sync_copy(deg2d.at[pl.ds(0, OWN // 16)], deg_out.at[w])
    pltpu.sync_copy(cntv, cnt_out.at[w])


def _prep_call(src, dst):
    deg, sbsl, cnt = pl.kernel(
        _prep_body,
        out_type=(
            jax.ShapeDtypeStruct((NW, OWN // 16, 16), jnp.float32),
            jax.ShapeDtypeStruct((NW, NSTRIP, 2, SELCAP), jnp.int32),
            jax.ShapeDtypeStruct((NW, NSTRIP * 16), jnp.int32),
        ),
        mesh=_mesh(),
        scratch_types=[
            pltpu.VMEM((STRIP,), jnp.int32),
            pltpu.VMEM((STRIP,), jnp.int32),
            pltpu.VMEM((SELCAP,), jnp.int32),
            pltpu.VMEM((SELCAP,), jnp.int32),
            pltpu.VMEM((OWN // 16 + 1, 16), jnp.float32),
            pltpu.VMEM((NSTRIP * 16,), jnp.int32),
            pltpu.SemaphoreType.DMA,
        ],
        compiler_params=_SC_PARAMS,
    )(src, dst)
    return deg.reshape(NPADR), sbsl, cnt


def _agg_body(hp_hbm, sbsl_hbm, cnt_hbm, out_hbm,
              selA, selB, cntv, gbufA, gbufB, acc,
              semLA, semLB, semGA, semGB):
    w = _wid()
    lo = w * OWN

    pltpu.sync_copy(hp_hbm.at[pl.ds(lo, OWN)], acc.at[pl.ds(0, OWN)])
    pltpu.sync_copy(cnt_hbm.at[w], cntv)

    iota16 = lax.iota(jnp.int32, 16)
    perms = [(iota16 + sh) % 16 for sh in range(16)]

    def start_list(sel, semL, s):
        pltpu.async_copy(sbsl_hbm.at[w, s], sel, semL)

    def wait_list(sel, semL):
        pltpu.make_async_copy(sbsl_hbm.at[w, 0], sel, semL).wait()

    def start_g0(sel, gbuf, semG):
        pltpu.async_copy(hp_hbm.at[sel.at[0, pl.ds(0, CH)]], gbuf, semG)

    def wait_g0(gbuf, semG):
        pltpu.make_async_copy(hp_hbm.at[pl.ds(0, CH)], gbuf, semG).wait()

    def acc_chunk(c2, sel, gbuf):
        def egroup(g, c3):
            dlv = jnp.clip(sel[1, pl.ds(c2 * CH + g * 16, 16)], 0, DUMP)
            jvec = g * 16 + iota16

            def cblk(cb, c4):
                for sh in range(16):
                    colv = cb * 16 + perms[sh]
                    vals = plsc.load_gather(gbuf, [jvec, colv])
                    plsc.addupdate_scatter(acc, [dlv, colv], vals)
                return c4
            lax.fori_loop(0, D // 16, cblk, 0)
            return c3
        lax.fori_loop(0, CH // 16, egroup, 0)

    def process(s, sel, gbuf, semG):
        m = cntv[pl.ds(s * 16, 16)][0]
        nch = (m + (CH - 1)) // CH
        wait_g0(gbuf, semG)
        acc_chunk(0, sel, gbuf)

        def chunk(c2, carry):
            pltpu.async_copy(hp_hbm.at[sel.at[0, pl.ds(c2 * CH, CH)]],
                             gbuf, semG).wait()
            acc_chunk(c2, sel, gbuf)
            return carry
        lax.fori_loop(1, nch, chunk, 0)

    start_list(selA, semLA, 0)
    wait_list(selA, semLA)
    start_list(selB, semLB, 1)
    start_g0(selA, gbufA, semGA)

    def pair(s2, c):
        s = 2 * s2
        process(s, selA, gbufA, semGA)

        @pl.when(s + 2 < NSTRIP)
        def _():
            start_list(selA, semLA, s + 2)

        wait_list(selB, semLB)
        start_g0(selB, gbufB, semGB)
        process(s + 1, selB, gbufB, semGB)

        @pl.when(s + 3 < NSTRIP)
        def _():
            start_list(selB, semLB, s + 3)

        @pl.when(s + 2 < NSTRIP)
        def _():
            wait_list(selA, semLA)
            start_g0(selA, gbufA, semGA)
        return c
    lax.fori_loop(0, NSTRIP // 2, pair, 0)

    pltpu.sync_copy(acc.at[pl.ds(0, OWN)], out_hbm.at[pl.ds(lo, OWN)])


def _agg_call(hp_pad, sbsl, cnt):
    return pl.kernel(
        _agg_body,
        out_type=jax.ShapeDtypeStruct((NPADR, D), jnp.float32),
        mesh=_mesh(),
        scratch_types=[
            pltpu.VMEM((2, SELCAP), jnp.int32),
            pltpu.VMEM((2, SELCAP), jnp.int32),
            pltpu.VMEM((NSTRIP * 16,), jnp.int32),
            pltpu.VMEM((CH, D), jnp.float32),
            pltpu.VMEM((CH, D), jnp.float32),
            pltpu.VMEM((OWN + 8, D), jnp.float32),
            pltpu.SemaphoreType.DMA,
            pltpu.SemaphoreType.DMA,
            pltpu.SemaphoreType.DMA,
            pltpu.SemaphoreType.DMA,
        ],
        compiler_params=_SC_PARAMS,
    )(hp_pad, sbsl, cnt)


RB = 1000
NRB = N // RB


def _dinv_block(deg_ref):
    return lax.rsqrt(deg_ref[...])


def _t0_body(x_ref, w_ref, deg_ref, out_ref):
    out_ref[...] = jnp.dot(x_ref[...], w_ref[...],
                           preferred_element_type=jnp.float32) * _dinv_block(deg_ref)


def _t0_call(x, W1, deg):
    return pl.pallas_call(
        _t0_body,
        grid=(NRB,),
        in_specs=[
            pl.BlockSpec((RB, D), lambda r: (r, 0)),
            pl.BlockSpec((D, D), lambda r: (0, 0)),
            pl.BlockSpec((RB, 1), lambda r: (r, 0)),
        ],
        out_specs=pl.BlockSpec((RB, D), lambda r: (r, 0)),
        out_shape=jax.ShapeDtypeStruct((N, D), jnp.float32),
    )(x, W1, deg)


def _stats_body(agg_ref, deg_ref, c_out, st_out, acc):
    r = pl.program_id(0)

    @pl.when(r == 0)
    def _():
        acc[...] = jnp.zeros_like(acc)

    cb = _dinv_block(deg_ref) * agg_ref[...]
    c_out[...] = cb
    acc[0:1, :] += jnp.sum(cb, axis=0, keepdims=True)
    acc[1:2, :] += jnp.sum(cb * cb, axis=0, keepdims=True)

    @pl.when(r == NRB - 1)
    def _():
        mean = acc[0:1, :] * (1.0 / N)
        var = acc[1:2, :] * (1.0 / N) - mean * mean
        st_out[0:1, :] = mean
        st_out[1:2, :] = lax.rsqrt(var + EPS)


def _stats_call(agg, deg):
    return pl.pallas_call(
        _stats_body,
        grid=(NRB,),
        in_specs=[
            pl.BlockSpec((RB, D), lambda r: (r, 0)),
            pl.BlockSpec((RB, 1), lambda r: (r, 0)),
        ],
        out_specs=[
            pl.BlockSpec((RB, D), lambda r: (r, 0)),
            pl.BlockSpec((2, D), lambda r: (0, 0)),
        ],
        out_shape=[
            jax.ShapeDtypeStruct((N, D), jnp.float32),
            jax.ShapeDtypeStruct((2, D), jnp.float32),
        ],
        scratch_shapes=[pltpu.VMEM((2, D), jnp.float32)],
    )(agg, deg)


def _t1b_body(c_ref, st_ref, g_ref, b_ref, w_ref, deg_ref, out_ref):
    y = jnp.maximum(
        g_ref[...] * (c_ref[...] - st_ref[0:1, :]) * st_ref[1:2, :] + b_ref[...],
        0.0)
    out_ref[...] = jnp.dot(y, w_ref[...],
                           preferred_element_type=jnp.float32) * _dinv_block(deg_ref)


def _t1b_call(c1, st1, g, b, W2, deg):
    return pl.pallas_call(
        _t1b_body,
        grid=(NRB,),
        in_specs=[
            pl.BlockSpec((RB, D), lambda r: (r, 0)),
            pl.BlockSpec((2, D), lambda r: (0, 0)),
            pl.BlockSpec((1, D), lambda r: (0, 0)),
            pl.BlockSpec((1, D), lambda r: (0, 0)),
            pl.BlockSpec((D, D), lambda r: (0, 0)),
            pl.BlockSpec((RB, 1), lambda r: (r, 0)),
        ],
        out_specs=pl.BlockSpec((RB, D), lambda r: (r, 0)),
        out_shape=jax.ShapeDtypeStruct((N, D), jnp.float32),
    )(c1, st1, g, b, W2, deg)


def _t2b_body(c_ref, st_ref, g_ref, b_ref, x_ref, out_ref):
    out_ref[...] = jnp.maximum(
        g_ref[...] * (c_ref[...] - st_ref[0:1, :]) * st_ref[1:2, :]
        + b_ref[...] + x_ref[...],
        0.0)


def _t2b_call(c2, st2, g, b, x):
    return pl.pallas_call(
        _t2b_body,
        grid=(NRB,),
        in_specs=[
            pl.BlockSpec((RB, D), lambda r: (r, 0)),
            pl.BlockSpec((2, D), lambda r: (0, 0)),
            pl.BlockSpec((1, D), lambda r: (0, 0)),
            pl.BlockSpec((1, D), lambda r: (0, 0)),
            pl.BlockSpec((RB, D), lambda r: (r, 0)),
        ],
        out_specs=pl.BlockSpec((RB, D), lambda r: (r, 0)),
        out_shape=jax.ShapeDtypeStruct((N, D), jnp.float32),
    )(c2, st2, g, b, x)


def kernel(x, edge_index, W1, b1, bn1_w, bn1_b, W2, b2, bn2_w, bn2_b):
    src = edge_index[0]
    dst = edge_index[1]
    deg, sbsl, cnt = _prep_call(src, dst)
    deg = deg.reshape(NPADR, 1)
    h1p = _t0_call(x, W1, deg)
    agg1 = _agg_call(jnp.pad(h1p, ((0, NPADR - N), (0, 0))), sbsl, cnt)
    c1, st1 = _stats_call(agg1, deg)
    h2p = _t1b_call(c1, st1, bn1_w.reshape(1, D), bn1_b.reshape(1, D), W2, deg)
    agg2 = _agg_call(jnp.pad(h2p, ((0, NPADR - N), (0, 0))), sbsl, cnt)
    c2, st2 = _stats_call(agg2, deg)
    return _t2b_call(c2, st2, bn2_w.reshape(1, D), bn2_b.reshape(1, D), x)
